# baseline (speedup 1.0000x reference)
import jax
import jax.numpy as jnp
from jax import lax
from jax.experimental import pallas as pl
from jax.experimental.pallas import tpu as pltpu

BATCH, BDASH, VOCAB, T, DMODEL = 16, 8, 32768, 4, 1024
NEG_INF = float("-inf")


def _body(lp_ref, bls_ref, seq_ref, st_ref, fid_ref, bsl_hbm, lp_hbm,
          bls_out, seq_out, st_out, big_out, beam_smem, sem):
    b = pl.program_id(0)
    bls_col = bls_ref[0]
    cand = lp_ref[...] + bls_col
    fid = fid_ref[...]
    iota_col = lax.broadcasted_iota(jnp.int32, (BDASH, 1), 0)
    lane8 = lax.broadcasted_iota(jnp.int32, (1, 1, BDASH), 2)
    seq_slab = seq_ref[...]
    seq_mask_iota = lax.broadcasted_iota(jnp.int32, (1, BDASH, T), 1)
    st_blk = st_ref[...]
    st_iota = lax.broadcasted_iota(jnp.int32, (2, BDASH, DMODEL), 1)
    big = jnp.int32(2 ** 30)

    def emit(k, nb, m, idx):
        beam_k = lax.shift_right_logical(idx, 15)
        sel_k = jnp.bitwise_and(idx, VOCAB - 1)
        beam_smem[k] = beam_k
        blsv = jnp.sum(jnp.where(iota_col == beam_k, bls_col, 0.0))
        nb = jnp.where(lane8 == k, blsv + m, nb)
        row = jnp.sum(jnp.where(seq_mask_iota == beam_k, seq_slab, 0),
                      axis=(0, 1))
        row5 = jnp.concatenate([row, jnp.broadcast_to(sel_k, (1,))], axis=0)
        seq_out[0, k, :] = row5
        st_out[:, k, :] = jnp.sum(
            jnp.where(st_iota == beam_k, st_blk, 0.0), axis=1)
        return nb

    v, iv = cand, fid
    n = VOCAB
    while n > 128:
        h = n // 2
        a, bb = v[:, :h], v[:, h:]
        ge = a >= bb
        v = jnp.where(ge, a, bb)
        iv = jnp.where(ge, iv[:, :h], iv[:, h:])
        n = h

    nb = jnp.zeros((1, 1, BDASH), jnp.float32)
    ms, ids = [], []
    for k in range(BDASH):
        m = jnp.max(v)
        idx = jnp.min(jnp.where(v == m, iv, big))
        ms.append(m)
        ids.append(idx)
        v = jnp.where((v == m) & (iv == idx), NEG_INF, v)
        nb = emit(k, nb, m, idx)
    bls_out[...] = nb

    def make_copies():
        cs = []
        for k in range(BDASH):
            src = b * BDASH + beam_smem[k]
            cs.append(pltpu.make_async_copy(
                bsl_hbm.at[src], big_out.at[0, k, 0:T, :], sem))
            cs.append(pltpu.make_async_copy(
                lp_hbm.at[pl.ds(src, 1), :],
                big_out.at[0, k, pl.ds(T, 1), :], sem))
        return cs

    copies = make_copies()
    for c in copies:
        c.start()

    v8, id8 = ms[BDASH - 1], ids[BDASH - 1]
    pred = (cand > v8) | ((cand == v8) & (fid <= id8))
    cnt = jnp.sum(pred.astype(jnp.int32))

    @pl.when(cnt != BDASH)
    def _slow_path():
        vals = cand
        nb = jnp.zeros((1, 1, BDASH), jnp.float32)
        for k in range(BDASH):
            m = jnp.max(vals)
            idx = jnp.min(jnp.where(vals == m, fid, big))
            nb = emit(k, nb, m, idx)
            vals = jnp.where(fid == idx, NEG_INF, vals)
        bls_out[...] = nb
        for c in copies:
            c.wait()
        redo = make_copies()
        for c in redo:
            c.start()

    for c in copies:
        c.wait()


def kernel(logprobs, beam_logprobs_sum, beam_seq, beam_seq_logprobs, state,
           beam_size):
    del beam_size
    nrows = BATCH * BDASH
    lp2 = logprobs.reshape(nrows, VOCAB)

    new_bls3, new_seq, new_state, new_bsl = pl.pallas_call(
        _body,
        grid=(BATCH,),
        in_specs=[
            pl.BlockSpec((BDASH, VOCAB), lambda b: (b, 0)),
            pl.BlockSpec((1, BDASH, 1), lambda b: (b, 0, 0)),
            pl.BlockSpec((1, BDASH, T), lambda b: (b, 0, 0)),
            pl.BlockSpec((2, BDASH, DMODEL), lambda b: (0, b, 0)),
            pl.BlockSpec((BDASH, VOCAB), lambda b: (0, 0)),
            pl.BlockSpec(memory_space=pltpu.MemorySpace.HBM),
            pl.BlockSpec(memory_space=pltpu.MemorySpace.HBM),
        ],
        out_specs=[
            pl.BlockSpec((1, 1, BDASH), lambda b: (b, 0, 0)),
            pl.BlockSpec((1, BDASH, T + 1), lambda b: (b, 0, 0)),
            pl.BlockSpec((2, BDASH, DMODEL), lambda b: (0, b, 0)),
            pl.BlockSpec((1, BDASH, T + 1, VOCAB), lambda b: (b, 0, 0, 0)),
        ],
        out_shape=[
            jax.ShapeDtypeStruct((BATCH, 1, BDASH), jnp.float32),
            jax.ShapeDtypeStruct((BATCH, BDASH, T + 1), jnp.int32),
            jax.ShapeDtypeStruct((2, nrows, DMODEL), jnp.float32),
            jax.ShapeDtypeStruct((BATCH, BDASH, T + 1, VOCAB), jnp.float32),
        ],
        scratch_shapes=[
            pltpu.SMEM((BDASH,), jnp.int32),
            pltpu.SemaphoreType.DMA,
        ],
    )(
        lp2,
        beam_logprobs_sum.reshape(BATCH, BDASH, 1),
        beam_seq,
        state,
        (jnp.arange(BDASH, dtype=jnp.int32)[:, None] * VOCAB
         + jnp.arange(VOCAB, dtype=jnp.int32)[None, :]),
        beam_seq_logprobs.reshape(nrows, T, VOCAB),
        lp2,
    )

    new_beam_logprobs_sum = new_bls3.reshape(BATCH, BDASH)
    return new_seq, new_bsl, new_beam_logprobs_sum, new_state


# speedup vs baseline: 1.0069x; 1.0069x over previous
import jax
import jax.numpy as jnp
from jax import lax
from jax.experimental import pallas as pl
from jax.experimental.pallas import tpu as pltpu

BATCH, BDASH, VOCAB, T, DMODEL = 16, 8, 32768, 4, 1024
NEG_INF = float("-inf")


def _body(lp_ref, bls_ref, seq_ref, st_ref, bsl_hbm, lp_hbm,
          bls_out, seq_out, st_out, big_out, beam_smem, sem):
    b = pl.program_id(0)
    bls_col = bls_ref[0]
    cand = lp_ref[...] + bls_col
    fid = (lax.broadcasted_iota(jnp.int32, (BDASH, VOCAB), 0) * VOCAB
           + lax.broadcasted_iota(jnp.int32, (BDASH, VOCAB), 1))
    iota_col = lax.broadcasted_iota(jnp.int32, (BDASH, 1), 0)
    lane8 = lax.broadcasted_iota(jnp.int32, (1, 1, BDASH), 2)
    seq_slab = seq_ref[...]
    seq_mask_iota = lax.broadcasted_iota(jnp.int32, (1, BDASH, T), 1)
    st_blk = st_ref[...]
    st_iota = lax.broadcasted_iota(jnp.int32, (2, BDASH, DMODEL), 1)
    big = jnp.int32(2 ** 30)

    def emit(k, nb, m, idx):
        beam_k = lax.shift_right_logical(idx, 15)
        sel_k = jnp.bitwise_and(idx, VOCAB - 1)
        beam_smem[k] = beam_k
        blsv = jnp.sum(jnp.where(iota_col == beam_k, bls_col, 0.0))
        nb = jnp.where(lane8 == k, blsv + m, nb)
        row = jnp.sum(jnp.where(seq_mask_iota == beam_k, seq_slab, 0),
                      axis=(0, 1))
        row5 = jnp.concatenate([row, jnp.broadcast_to(sel_k, (1,))], axis=0)
        seq_out[0, k, :] = row5
        st_out[:, k, :] = jnp.sum(
            jnp.where(st_iota == beam_k, st_blk, 0.0), axis=1)
        return nb

    v, iv = cand, fid
    n = VOCAB
    while n > 128:
        h = n // 2
        a, bb = v[:, :h], v[:, h:]
        ge = a >= bb
        v = jnp.where(ge, a, bb)
        iv = jnp.where(ge, iv[:, :h], iv[:, h:])
        n = h

    nb = jnp.zeros((1, 1, BDASH), jnp.float32)
    ms, ids = [], []
    for k in range(BDASH):
        m = jnp.max(v)
        idx = jnp.min(jnp.where(v == m, iv, big))
        ms.append(m)
        ids.append(idx)
        v = jnp.where((v == m) & (iv == idx), NEG_INF, v)
        nb = emit(k, nb, m, idx)
    bls_out[...] = nb

    def make_copies():
        cs = []
        for k in range(BDASH):
            src = b * BDASH + beam_smem[k]
            cs.append(pltpu.make_async_copy(
                bsl_hbm.at[src], big_out.at[0, k, 0:T, :], sem))
            cs.append(pltpu.make_async_copy(
                lp_hbm.at[pl.ds(src, 1), :],
                big_out.at[0, k, pl.ds(T, 1), :], sem))
        return cs

    copies = make_copies()
    for c in copies:
        c.start()

    v8, id8 = ms[BDASH - 1], ids[BDASH - 1]
    pred = (cand > v8) | ((cand == v8) & (fid <= id8))
    cnt = jnp.sum(pred.astype(jnp.int32))

    @pl.when(cnt != BDASH)
    def _slow_path():
        vals = cand
        nb = jnp.zeros((1, 1, BDASH), jnp.float32)
        for k in range(BDASH):
            m = jnp.max(vals)
            idx = jnp.min(jnp.where(vals == m, fid, big))
            nb = emit(k, nb, m, idx)
            vals = jnp.where(fid == idx, NEG_INF, vals)
        bls_out[...] = nb
        for c in copies:
            c.wait()
        redo = make_copies()
        for c in redo:
            c.start()

    for c in copies:
        c.wait()


def kernel(logprobs, beam_logprobs_sum, beam_seq, beam_seq_logprobs, state,
           beam_size):
    del beam_size
    nrows = BATCH * BDASH
    lp2 = logprobs.reshape(nrows, VOCAB)

    new_bls3, new_seq, new_state, new_bsl = pl.pallas_call(
        _body,
        grid=(BATCH,),
        in_specs=[
            pl.BlockSpec((BDASH, VOCAB), lambda b: (b, 0)),
            pl.BlockSpec((1, BDASH, 1), lambda b: (b, 0, 0)),
            pl.BlockSpec((1, BDASH, T), lambda b: (b, 0, 0)),
            pl.BlockSpec((2, BDASH, DMODEL), lambda b: (0, b, 0)),
            pl.BlockSpec(memory_space=pltpu.MemorySpace.HBM),
            pl.BlockSpec(memory_space=pltpu.MemorySpace.HBM),
        ],
        out_specs=[
            pl.BlockSpec((1, 1, BDASH), lambda b: (b, 0, 0)),
            pl.BlockSpec((1, BDASH, T + 1), lambda b: (b, 0, 0)),
            pl.BlockSpec((2, BDASH, DMODEL), lambda b: (0, b, 0)),
            pl.BlockSpec((1, BDASH, T + 1, VOCAB), lambda b: (b, 0, 0, 0)),
        ],
        out_shape=[
            jax.ShapeDtypeStruct((BATCH, 1, BDASH), jnp.float32),
            jax.ShapeDtypeStruct((BATCH, BDASH, T + 1), jnp.int32),
            jax.ShapeDtypeStruct((2, nrows, DMODEL), jnp.float32),
            jax.ShapeDtypeStruct((BATCH, BDASH, T + 1, VOCAB), jnp.float32),
        ],
        scratch_shapes=[
            pltpu.SMEM((BDASH,), jnp.int32),
            pltpu.SemaphoreType.DMA,
        ],
    )(
        lp2,
        beam_logprobs_sum.reshape(BATCH, BDASH, 1),
        beam_seq,
        state,
        beam_seq_logprobs.reshape(nrows, T, VOCAB),
        lp2,
    )

    new_beam_logprobs_sum = new_bls3.reshape(BATCH, BDASH)
    return new_seq, new_bsl, new_beam_logprobs_sum, new_state
